# Initial kernel scaffold; baseline (speedup 1.0000x reference)
import jax
import jax.numpy as jnp
from jax.experimental import pallas as pl


def kernel(x, feature_mask, w, W1, b1, W2, b2, W3, b3):
    raise NotImplementedError("write your pallas kernel here")


# speedup vs baseline: 2.8045x; 2.8045x over previous
import jax
import jax.numpy as jnp
from jax.experimental import pallas as pl
from functools import partial

N, F, H, C = 50000, 512, 64, 16
BLOCK_N = 2000


def _fused_kernel(x_ref, mask_ref, w_ref, w1t_ref, b1_ref, w2t_ref, b2_ref,
                  w3t_ref, b3_ref, out_ref):
    wm = w_ref[...] * mask_ref[...]
    h = jnp.dot(x_ref[...], wm, preferred_element_type=jnp.float32)
    h = jnp.maximum(
        jnp.dot(h, w1t_ref[...], preferred_element_type=jnp.float32)
        + b1_ref[...], 0.0)
    h = jnp.maximum(
        jnp.dot(h, w2t_ref[...], preferred_element_type=jnp.float32)
        + b2_ref[...], 0.0)
    out_ref[...] = (
        jnp.dot(h, w3t_ref[...], preferred_element_type=jnp.float32)
        + b3_ref[...])


@jax.jit
def kernel(x, feature_mask, w, W1, b1, W2, b2, W3, b3):
    mask_f = feature_mask.astype(jnp.float32).reshape(F, 1)
    grid = (N + BLOCK_N - 1) // BLOCK_N
    full = lambda *s: pl.BlockSpec(s, lambda i: (0,) * len(s))
    return pl.pallas_call(
        _fused_kernel,
        grid=(grid,),
        in_specs=[
            pl.BlockSpec((BLOCK_N, F), lambda i: (i, 0)),
            full(F, 1),
            full(F, H),
            full(H, H),
            full(1, H),
            full(H, H),
            full(1, H),
            full(H, C),
            full(1, C),
        ],
        out_specs=pl.BlockSpec((BLOCK_N, C), lambda i: (i, 0)),
        out_shape=jax.ShapeDtypeStruct((N, C), jnp.float32),
    )(x, mask_f, w, W1.T, b1.reshape(1, H), W2.T, b2.reshape(1, H),
      W3.T, b3.reshape(1, C))


# speedup vs baseline: 2.9007x; 1.0343x over previous
import jax
import jax.numpy as jnp
from jax.experimental import pallas as pl
from functools import partial

N, F, H, C = 50000, 512, 64, 16
BLOCK_N = 2000


def _fused_kernel(x_ref, mask_ref, w_ref, w1t_ref, b1_ref, w2t_ref, b2_ref,
                  w3t_ref, b3_ref, out_ref):
    wm = w_ref[...] * mask_ref[...]
    wc = jnp.dot(wm, w1t_ref[...], preferred_element_type=jnp.float32)
    h = jnp.maximum(
        jnp.dot(x_ref[...], wc, preferred_element_type=jnp.float32)
        + b1_ref[...], 0.0)
    h = jnp.maximum(
        jnp.dot(h, w2t_ref[...], preferred_element_type=jnp.float32)
        + b2_ref[...], 0.0)
    out_ref[...] = (
        jnp.dot(h, w3t_ref[...], preferred_element_type=jnp.float32)
        + b3_ref[...])


@jax.jit
def kernel(x, feature_mask, w, W1, b1, W2, b2, W3, b3):
    mask_f = feature_mask.astype(jnp.float32).reshape(F, 1)
    grid = (N + BLOCK_N - 1) // BLOCK_N
    full = lambda *s: pl.BlockSpec(s, lambda i: (0,) * len(s))
    return pl.pallas_call(
        _fused_kernel,
        grid=(grid,),
        in_specs=[
            pl.BlockSpec((BLOCK_N, F), lambda i: (i, 0)),
            full(F, 1),
            full(F, H),
            full(H, H),
            full(1, H),
            full(H, H),
            full(1, H),
            full(H, C),
            full(1, C),
        ],
        out_specs=pl.BlockSpec((BLOCK_N, C), lambda i: (i, 0)),
        out_shape=jax.ShapeDtypeStruct((N, C), jnp.float32),
    )(x, mask_f, w, W1.T, b1.reshape(1, H), W2.T, b2.reshape(1, H),
      W3.T, b3.reshape(1, C))
